# baseline (speedup 1.0000x reference)
import jax
import jax.numpy as jnp
from jax.experimental import pallas as pl
from jax.experimental.pallas import tpu as pltpu

B, T, L, F = 16, 32, 4, 1024
OUT_POINTS = 4096
DIM = 3
M = B * T
K = L * F
N = OUT_POINTS * DIM

BN = 1024


def _matmul_body(x_ref, w_ref, b_ref, o_ref):
    acc = jnp.broadcast_to(b_ref[...], o_ref.shape)
    for l in range(L):
        acc += jnp.dot(x_ref[:, (L - 1 - l) * F:(L - l) * F],
                       w_ref[l * F:(l + 1) * F, :],
                       preferred_element_type=jnp.float32)
    o_ref[...] = acc


@jax.jit
def _decode(rnn, W, b):
    x = rnn.reshape(M, K)
    b2 = b.reshape(1, N)

    out = pl.pallas_call(
        _matmul_body,
        grid=(N // BN,),
        in_specs=[
            pl.BlockSpec((M, K), lambda j: (0, 0)),
            pl.BlockSpec((K, BN), lambda j: (0, j)),
            pl.BlockSpec((1, BN), lambda j: (0, j)),
        ],
        out_specs=pl.BlockSpec((M, BN), lambda j: (0, j)),
        out_shape=jax.ShapeDtypeStruct((M, N), jnp.float32),
        compiler_params=pltpu.CompilerParams(
            dimension_semantics=("arbitrary",),
        ),
    )(x, W, b2)
    return out.reshape(B, T, OUT_POINTS, DIM)


def kernel(rnn, enc_xyz, enc_feats, W, b):
    del enc_xyz, enc_feats
    return _decode(rnn, W, b)


# speedup vs baseline: 1.0180x; 1.0180x over previous
import jax
import jax.numpy as jnp
from jax.experimental import pallas as pl
from jax.experimental.pallas import tpu as pltpu

B, T, L, F = 16, 32, 4, 1024
OUT_POINTS = 4096
DIM = 3
M = B * T
K = L * F
N = OUT_POINTS * DIM

BN = 1024


def _matmul_body(x_ref, w_ref, b_ref, o_ref):
    acc = jnp.broadcast_to(b_ref[...], o_ref.shape)
    for l in range(L):
        acc += jnp.dot(x_ref[:, (L - 1 - l) * F:(L - l) * F],
                       w_ref[l * F:(l + 1) * F, :].astype(jnp.bfloat16),
                       preferred_element_type=jnp.float32)
    o_ref[...] = acc


@jax.jit
def _decode(rnn, W, b):
    x = rnn.reshape(M, K).astype(jnp.bfloat16)
    b2 = b.reshape(1, N)

    out = pl.pallas_call(
        _matmul_body,
        grid=(N // BN,),
        in_specs=[
            pl.BlockSpec((M, K), lambda j: (0, 0)),
            pl.BlockSpec((K, BN), lambda j: (0, j)),
            pl.BlockSpec((1, BN), lambda j: (0, j)),
        ],
        out_specs=pl.BlockSpec((M, BN), lambda j: (0, j)),
        out_shape=jax.ShapeDtypeStruct((M, N), jnp.float32),
        compiler_params=pltpu.CompilerParams(
            dimension_semantics=("arbitrary",),
        ),
    )(x, W, b2)
    return out.reshape(B, T, OUT_POINTS, DIM)


def kernel(rnn, enc_xyz, enc_feats, W, b):
    del enc_xyz, enc_feats
    return _decode(rnn, W, b)
